# baseline (speedup 1.0000x reference)
import functools

import numpy as np

import jax
import jax.numpy as jnp
from jax import lax
from jax.experimental import pallas as pl
from jax.experimental.pallas import tpu as pltpu
from jax.experimental.pallas import tpu_sc as plsc

VOCAB = 151936
D_MODEL = 2048
HEAD_DIM = 128
ROPE_THETA = 1000000.0
B = 2
S = 4096
TOK = B * S
NC, NS = 2, 16
NW = NC * NS
TOK_PER_W = TOK // NW
CHUNK = 16
NCHUNK = TOK_PER_W // CHUNK

MIN_F32 = float(jnp.finfo(jnp.float32).min)
BAND = 512
NBAND = S // BAND

_INV_FREQ = 1.0 / (ROPE_THETA ** (
    np.arange(0, HEAD_DIM, 2, dtype=np.float32) / HEAD_DIM))
_INV_FULL = np.concatenate([_INV_FREQ, _INV_FREQ])[None, :]


def _sc_gather(ids, table):
    mesh = plsc.VectorSubcoreMesh(
        core_axis_name="c", subcore_axis_name="s",
        num_cores=NC, num_subcores=NS)

    @functools.partial(
        pl.kernel,
        out_type=jax.ShapeDtypeStruct((TOK, D_MODEL), jnp.float32),
        mesh=mesh,
        scratch_types=[
            pltpu.VMEM((TOK_PER_W,), jnp.int32),
            pltpu.VMEM((CHUNK, D_MODEL), jnp.float32),
            pltpu.VMEM((CHUNK, D_MODEL), jnp.float32),
            pltpu.SemaphoreType.DMA,
            pltpu.SemaphoreType.DMA,
            pltpu.SemaphoreType.DMA,
            pltpu.SemaphoreType.DMA,
        ],
    )
    def gather_kernel(ids_hbm, table_hbm, out_hbm, idx_v, buf0, buf1,
                      g0, g1, s0, s1):
        wid = lax.axis_index("s") * NC + lax.axis_index("c")
        base = wid * TOK_PER_W
        b = wid // (NW // B)
        off = (wid % (NW // B)) * TOK_PER_W
        pltpu.sync_copy(ids_hbm.at[b, pl.ds(off, TOK_PER_W)], idx_v)
        bufs = (buf0, buf1)
        gsems = (g0, g1)
        ssems = (s0, s1)

        def gather_start(c):
            return pltpu.async_copy(
                table_hbm.at[idx_v.at[pl.ds(c * CHUNK, CHUNK)]],
                bufs[c % 2], gsems[c % 2])

        def store_start(c):
            return pltpu.async_copy(
                bufs[c % 2], out_hbm.at[pl.ds(base + c * CHUNK, CHUNK)],
                ssems[c % 2])

        g = gather_start(0)
        stores = [None, None]
        for c in range(NCHUNK):
            if c + 1 < NCHUNK:
                if stores[(c + 1) % 2] is not None:
                    stores[(c + 1) % 2].wait()
                g_next = gather_start(c + 1)
            g.wait()
            stores[c % 2] = store_start(c)
            if c + 1 < NCHUNK:
                g = g_next
        stores[0].wait()
        stores[1].wait()

    return gather_kernel(ids, table)


def _const_body(invf_ref, mask_ref, cos_ref, sin_ref, pos_ref):
    i = pl.program_id(0)
    row = i * BAND + lax.broadcasted_iota(jnp.int32, (BAND, S), 0)
    col = lax.broadcasted_iota(jnp.int32, (BAND, S), 1)
    m = jnp.where(col <= row, jnp.float32(0.0), jnp.float32(MIN_F32))
    mask_ref[0, 0] = m
    mask_ref[1, 0] = m
    pos = i * BAND + lax.broadcasted_iota(jnp.int32, (BAND, HEAD_DIM), 0)
    freqs = pos.astype(jnp.float32) * invf_ref[...]
    cos_ref[0] = jnp.cos(freqs)
    sin_ref[0] = jnp.sin(freqs)
    pos_ref[...] = i * BAND + lax.broadcasted_iota(jnp.int32, (1, BAND), 1)


def _make_consts():
    return pl.pallas_call(
        _const_body,
        grid=(NBAND,),
        in_specs=[pl.BlockSpec((1, HEAD_DIM), lambda i: (0, 0))],
        out_specs=(
            pl.BlockSpec((B, 1, BAND, S), lambda i: (0, 0, i, 0)),
            pl.BlockSpec((1, BAND, HEAD_DIM), lambda i: (0, i, 0)),
            pl.BlockSpec((1, BAND, HEAD_DIM), lambda i: (0, i, 0)),
            pl.BlockSpec((1, BAND), lambda i: (0, i)),
        ),
        out_shape=(
            jax.ShapeDtypeStruct((B, 1, S, S), jnp.float32),
            jax.ShapeDtypeStruct((1, S, HEAD_DIM), jnp.float32),
            jax.ShapeDtypeStruct((1, S, HEAD_DIM), jnp.float32),
            jax.ShapeDtypeStruct((1, S), jnp.int32),
        ),
    )(jnp.asarray(_INV_FULL))


def kernel(input_ids, embed_table):
    embeds = _sc_gather(input_ids, embed_table).reshape(B, S, D_MODEL)
    causal_mask, cos, sin, position_ids = _make_consts()
    return (embeds, causal_mask, position_ids, cos, sin)


# speedup vs baseline: 1.0147x; 1.0147x over previous
import functools

import numpy as np

import jax
import jax.numpy as jnp
from jax import lax
from jax.experimental import pallas as pl
from jax.experimental.pallas import tpu as pltpu
from jax.experimental.pallas import tpu_sc as plsc

VOCAB = 151936
D_MODEL = 2048
HEAD_DIM = 128
ROPE_THETA = 1000000.0
B = 2
S = 4096
TOK = B * S
NC, NS = 2, 16
NW = NC * NS
TOK_PER_W = TOK // NW
CHUNK = 32
NCHUNK = TOK_PER_W // CHUNK

MIN_F32 = float(jnp.finfo(jnp.float32).min)
BAND = 256
NBAND = S // BAND

_INV_FREQ = 1.0 / (ROPE_THETA ** (
    np.arange(0, HEAD_DIM, 2, dtype=np.float32) / HEAD_DIM))
_INV_FULL = np.concatenate([_INV_FREQ, _INV_FREQ])[None, :]


def _sc_gather(ids, table):
    mesh = plsc.VectorSubcoreMesh(
        core_axis_name="c", subcore_axis_name="s",
        num_cores=NC, num_subcores=NS)

    @functools.partial(
        pl.kernel,
        out_type=jax.ShapeDtypeStruct((TOK, D_MODEL), jnp.float32),
        mesh=mesh,
        scratch_types=[
            pltpu.VMEM((TOK_PER_W,), jnp.int32),
            pltpu.VMEM((CHUNK, D_MODEL), jnp.float32),
            pltpu.SemaphoreType.DMA,
        ],
    )
    def gather_kernel(ids_hbm, table_hbm, out_hbm, idx_v, buf, gsem):
        wid = lax.axis_index("s") * NC + lax.axis_index("c")
        base = wid * TOK_PER_W
        b = wid // (NW // B)
        off = (wid % (NW // B)) * TOK_PER_W
        pltpu.sync_copy(ids_hbm.at[b, pl.ds(off, TOK_PER_W)], idx_v)

        def chunk_body(c, carry):
            start = pl.multiple_of(c * CHUNK, CHUNK)
            pltpu.async_copy(
                table_hbm.at[idx_v.at[pl.ds(start, CHUNK)]], buf, gsem
            ).wait()
            pltpu.sync_copy(buf, out_hbm.at[pl.ds(base + start, CHUNK)])
            return carry

        lax.fori_loop(0, NCHUNK, chunk_body, 0)

    return gather_kernel(ids, table)


def _const_body(invf_ref, mask_ref, cos_ref, sin_ref, pos_ref):
    i = pl.program_id(0)
    row = i * BAND + lax.broadcasted_iota(jnp.int32, (BAND, S), 0)
    col = lax.broadcasted_iota(jnp.int32, (BAND, S), 1)
    m = jnp.where(col <= row, jnp.float32(0.0), jnp.float32(MIN_F32))
    mask_ref[0, 0] = m
    mask_ref[1, 0] = m
    pos = i * BAND + lax.broadcasted_iota(jnp.int32, (BAND, HEAD_DIM), 0)
    freqs = pos.astype(jnp.float32) * invf_ref[...]
    cos_ref[0] = jnp.cos(freqs)
    sin_ref[0] = jnp.sin(freqs)
    pos_ref[...] = i * BAND + lax.broadcasted_iota(jnp.int32, (1, BAND), 1)


def _make_consts():
    return pl.pallas_call(
        _const_body,
        grid=(NBAND,),
        in_specs=[pl.BlockSpec((1, HEAD_DIM), lambda i: (0, 0))],
        out_specs=(
            pl.BlockSpec((B, 1, BAND, S), lambda i: (0, 0, i, 0)),
            pl.BlockSpec((1, BAND, HEAD_DIM), lambda i: (0, i, 0)),
            pl.BlockSpec((1, BAND, HEAD_DIM), lambda i: (0, i, 0)),
            pl.BlockSpec((1, BAND), lambda i: (0, i)),
        ),
        out_shape=(
            jax.ShapeDtypeStruct((B, 1, S, S), jnp.float32),
            jax.ShapeDtypeStruct((1, S, HEAD_DIM), jnp.float32),
            jax.ShapeDtypeStruct((1, S, HEAD_DIM), jnp.float32),
            jax.ShapeDtypeStruct((1, S), jnp.int32),
        ),
    )(jnp.asarray(_INV_FULL))


def kernel(input_ids, embed_table):
    embeds = _sc_gather(input_ids, embed_table).reshape(B, S, D_MODEL)
    causal_mask, cos, sin, position_ids = _make_consts()
    return (embeds, causal_mask, position_ids, cos, sin)


# speedup vs baseline: 1.0181x; 1.0033x over previous
import functools

import numpy as np

import jax
import jax.numpy as jnp
from jax import lax
from jax.experimental import pallas as pl
from jax.experimental.pallas import tpu as pltpu
from jax.experimental.pallas import tpu_sc as plsc

VOCAB = 151936
D_MODEL = 2048
HEAD_DIM = 128
ROPE_THETA = 1000000.0
B = 2
S = 4096
TOK = B * S
NC, NS = 2, 16
NW = NC * NS
TOK_PER_W = TOK // NW
CHUNK = 32
NCHUNK = TOK_PER_W // CHUNK

MIN_F32 = float(jnp.finfo(jnp.float32).min)
BAND = 128
NBAND = S // BAND

_INV_FREQ = 1.0 / (ROPE_THETA ** (
    np.arange(0, HEAD_DIM, 2, dtype=np.float32) / HEAD_DIM))
_INV_FULL = np.concatenate([_INV_FREQ, _INV_FREQ])[None, :]


def _sc_gather(ids, table):
    mesh = plsc.VectorSubcoreMesh(
        core_axis_name="c", subcore_axis_name="s",
        num_cores=NC, num_subcores=NS)

    @functools.partial(
        pl.kernel,
        out_type=jax.ShapeDtypeStruct((TOK, D_MODEL), jnp.float32),
        mesh=mesh,
        scratch_types=[
            pltpu.VMEM((TOK_PER_W,), jnp.int32),
            pltpu.VMEM((CHUNK, D_MODEL), jnp.float32),
            pltpu.SemaphoreType.DMA,
        ],
    )
    def gather_kernel(ids_hbm, table_hbm, out_hbm, idx_v, buf, gsem):
        wid = lax.axis_index("s") * NC + lax.axis_index("c")
        base = wid * TOK_PER_W
        b = wid // (NW // B)
        off = (wid % (NW // B)) * TOK_PER_W
        pltpu.sync_copy(ids_hbm.at[b, pl.ds(off, TOK_PER_W)], idx_v)

        def chunk_body(c, carry):
            start = pl.multiple_of(c * CHUNK, CHUNK)
            pltpu.async_copy(
                table_hbm.at[idx_v.at[pl.ds(start, CHUNK)]], buf, gsem
            ).wait()
            pltpu.sync_copy(buf, out_hbm.at[pl.ds(base + start, CHUNK)])
            return carry

        lax.fori_loop(0, NCHUNK, chunk_body, 0)

    return gather_kernel(ids, table)


def _const_body(invf_ref, mask_ref, cos_ref, sin_ref, pos_ref):
    i = pl.program_id(0)
    row = i * BAND + lax.broadcasted_iota(jnp.int32, (BAND, S), 0)
    col = lax.broadcasted_iota(jnp.int32, (BAND, S), 1)
    m = jnp.where(col <= row, jnp.float32(0.0), jnp.float32(MIN_F32))
    mask_ref[0, 0] = m
    mask_ref[1, 0] = m
    pos = i * BAND + lax.broadcasted_iota(jnp.int32, (BAND, HEAD_DIM), 0)
    freqs = pos.astype(jnp.float32) * invf_ref[...]
    cos_ref[0] = jnp.cos(freqs)
    sin_ref[0] = jnp.sin(freqs)
    pos_ref[...] = i * BAND + lax.broadcasted_iota(jnp.int32, (1, BAND), 1)


def _make_consts():
    return pl.pallas_call(
        _const_body,
        grid=(NBAND,),
        in_specs=[pl.BlockSpec((1, HEAD_DIM), lambda i: (0, 0))],
        out_specs=(
            pl.BlockSpec((B, 1, BAND, S), lambda i: (0, 0, i, 0)),
            pl.BlockSpec((1, BAND, HEAD_DIM), lambda i: (0, i, 0)),
            pl.BlockSpec((1, BAND, HEAD_DIM), lambda i: (0, i, 0)),
            pl.BlockSpec((1, BAND), lambda i: (0, i)),
        ),
        out_shape=(
            jax.ShapeDtypeStruct((B, 1, S, S), jnp.float32),
            jax.ShapeDtypeStruct((1, S, HEAD_DIM), jnp.float32),
            jax.ShapeDtypeStruct((1, S, HEAD_DIM), jnp.float32),
            jax.ShapeDtypeStruct((1, S), jnp.int32),
        ),
    )(jnp.asarray(_INV_FULL))


def kernel(input_ids, embed_table):
    embeds = _sc_gather(input_ids, embed_table).reshape(B, S, D_MODEL)
    causal_mask, cos, sin, position_ids = _make_consts()
    return (embeds, causal_mask, position_ids, cos, sin)
